# baseline (speedup 1.0000x reference)
import functools

import jax
import jax.numpy as jnp
from jax import lax
from jax.experimental import pallas as pl
from jax.experimental.pallas import tpu as pltpu
from jax.experimental.pallas import tpu_sc as plsc


def kernel(head_idx, rel_idx, tail_idx, entity_w, rel_w):
    B = head_idx.shape[0]
    D = entity_w.shape[1]

    info = plsc.get_sparse_core_info()
    NC, NS, L = info.num_cores, info.num_subcores, info.num_lanes
    NW = NC * NS

    bpw = B // NW
    CH = 128
    nch = bpw // CH
    ngrp = bpw // L
    assert bpw * NW == B and nch * CH == bpw and ngrp * L == bpw

    h_i = head_idx.astype(jnp.int32).reshape(NW * nch, CH)
    r_i = rel_idx.astype(jnp.int32).reshape(NW * nch, CH)
    t_i = tail_idx.astype(jnp.int32).reshape(NW * nch, CH)

    mesh = plsc.VectorSubcoreMesh(core_axis_name="c", subcore_axis_name="s")

    @functools.partial(
        pl.kernel,
        out_type=jax.ShapeDtypeStruct((B,), jnp.float32),
        mesh=mesh,
        compiler_params=pltpu.CompilerParams(
            needs_layout_passes=False, use_tc_tiling_on_sc=False
        ),
        scratch_types=[
            pltpu.VMEM((nch, CH), jnp.int32),
            pltpu.VMEM((nch, CH), jnp.int32),
            pltpu.VMEM((nch, CH), jnp.int32),
            pltpu.VMEM((bpw, D), jnp.float32),
            pltpu.VMEM((bpw, D), jnp.float32),
            pltpu.VMEM((bpw, D), jnp.float32),
            pltpu.VMEM((bpw,), jnp.float32),
            pltpu.SemaphoreType.DMA,
        ],
    )
    def distmult(ent_hbm, relw_hbm, hih, rih, tih, out_hbm,
                 hiv, riv, tiv, hv, rv, tv, ov, sem):
        wid = lax.axis_index("s") * NC + lax.axis_index("c")
        cb = wid * nch
        pltpu.sync_copy(hih.at[pl.ds(cb, nch)], hiv)
        pltpu.sync_copy(rih.at[pl.ds(cb, nch)], riv)
        pltpu.sync_copy(tih.at[pl.ds(cb, nch)], tiv)

        copies = []
        for j in range(nch):
            dst = pl.ds(j * CH, CH)
            copies.append(pltpu.async_copy(ent_hbm.at[hiv.at[j]], hv.at[dst], sem))
            copies.append(pltpu.async_copy(relw_hbm.at[riv.at[j]], rv.at[dst], sem))
            copies.append(pltpu.async_copy(ent_hbm.at[tiv.at[j]], tv.at[dst], sem))
        for c in copies:
            c.wait()

        def group(g, carry):
            rowv = g * L + lax.iota(jnp.int32, L)
            acc = jnp.zeros((L,), jnp.float32)
            for d in range(D):
                col = jnp.full((L,), d, jnp.int32)
                a = plsc.load_gather(hv, [rowv, col])
                b = plsc.load_gather(rv, [rowv, col])
                c = plsc.load_gather(tv, [rowv, col])
                acc = acc + a * b * c
            ov[pl.ds(g * L, L)] = 1.0 / (1.0 + jnp.exp(-acc))
            return carry

        lax.fori_loop(0, ngrp, group, 0)
        pltpu.sync_copy(ov, out_hbm.at[pl.ds(wid * bpw, bpw)])

    return distmult(entity_w, rel_w, h_i, r_i, t_i)


# speedup vs baseline: 1.0118x; 1.0118x over previous
import functools

import jax
import jax.numpy as jnp
from jax import lax
from jax.experimental import pallas as pl
from jax.experimental.pallas import tpu as pltpu
from jax.experimental.pallas import tpu_sc as plsc


def kernel(head_idx, rel_idx, tail_idx, entity_w, rel_w):
    B = head_idx.shape[0]
    D = entity_w.shape[1]
    NR = rel_w.shape[0]
    PW = 128

    info = plsc.get_sparse_core_info()
    NC, NS, L = info.num_cores, info.num_subcores, info.num_lanes
    NW = NC * NS

    bpw = B // NW
    CH = 128
    nch = bpw // CH
    gpc = CH // L
    assert bpw * NW == B and nch * CH == bpw and gpc * L == CH

    h_i = head_idx.astype(jnp.int32).reshape(NW * nch, CH)
    r_i = rel_idx.astype(jnp.int32).reshape(NW * nch, CH)
    t_i = tail_idx.astype(jnp.int32).reshape(NW * nch, CH)

    ent_p = jnp.pad(entity_w, ((0, 0), (0, PW - D)))

    mesh = plsc.VectorSubcoreMesh(core_axis_name="c", subcore_axis_name="s")

    @functools.partial(
        pl.kernel,
        out_type=jax.ShapeDtypeStruct((B,), jnp.float32),
        mesh=mesh,
        compiler_params=pltpu.CompilerParams(
            needs_layout_passes=False, use_tc_tiling_on_sc=False
        ),
        scratch_types=[
            pltpu.VMEM((nch, CH), jnp.int32),
            pltpu.VMEM((nch, CH), jnp.int32),
            pltpu.VMEM((nch, CH), jnp.int32),
            pltpu.VMEM((2, CH, PW), jnp.float32),
            pltpu.VMEM((2, CH, PW), jnp.float32),
            pltpu.VMEM((NR, D), jnp.float32),
            pltpu.VMEM((bpw,), jnp.float32),
            pltpu.SemaphoreType.DMA,
        ],
    )
    def distmult(ent_hbm, relw_hbm, hih, rih, tih, out_hbm,
                 hiv, riv, tiv, hv, tv, rv, ov, sem):
        wid = lax.axis_index("s") * NC + lax.axis_index("c")
        cb = wid * nch
        pltpu.sync_copy(hih.at[pl.ds(cb, nch)], hiv)
        pltpu.sync_copy(rih.at[pl.ds(cb, nch)], riv)
        pltpu.sync_copy(tih.at[pl.ds(cb, nch)], tiv)
        pltpu.sync_copy(relw_hbm, rv)

        def fire(j, buf):
            return (
                pltpu.async_copy(ent_hbm.at[hiv.at[j]], hv.at[buf], sem),
                pltpu.async_copy(ent_hbm.at[tiv.at[j]], tv.at[buf], sem),
            )

        inflight = fire(0, 0)
        for j in range(nch):
            buf = j % 2
            for c in inflight:
                c.wait()
            if j + 1 < nch:
                inflight = fire(j + 1, 1 - buf)

            ridx_chunk = riv.at[j]

            def group(g, carry):
                rowv = g * L + lax.iota(jnp.int32, L)
                ridx = ridx_chunk[pl.ds(g * L, L)]
                acc = jnp.zeros((L,), jnp.float32)
                for d in range(D):
                    col = jnp.full((L,), d, jnp.int32)
                    a = plsc.load_gather(hv.at[buf], [rowv, col])
                    b = plsc.load_gather(rv, [ridx, col])
                    c = plsc.load_gather(tv.at[buf], [rowv, col])
                    acc = acc + a * b * c
                ov[pl.ds(j * CH + g * L, L)] = 1.0 / (1.0 + jnp.exp(-acc))
                return carry

            lax.fori_loop(0, gpc, group, 0)

        pltpu.sync_copy(ov, out_hbm.at[pl.ds(wid * bpw, bpw)])

    return distmult(ent_p, rel_w, h_i, r_i, t_i)
